# Initial kernel scaffold; baseline (speedup 1.0000x reference)
import jax
import jax.numpy as jnp
from jax.experimental import pallas as pl


def kernel(sampling_map):
    raise NotImplementedError("write your pallas kernel here")


# speedup vs baseline: 1.3351x; 1.3351x over previous
import functools

import jax
import jax.numpy as jnp
from jax import lax
from jax.experimental import pallas as pl

GS = 8
CELL = 64
WGS = 3
WS = GS - WGS + 1
TOP_K = 4


def _pool_kernel(x_ref, out_ref):
    x = x_ref[0]
    f32 = jnp.float32
    c_i = lax.broadcasted_iota(jnp.int32, (512, GS), 0) // CELL
    j_i = lax.broadcasted_iota(jnp.int32, (512, GS), 1)
    p1 = (c_i == j_i).astype(f32)
    y = lax.dot_general(x, p1, (((1,), (0,)), ((), ())),
                        precision=lax.Precision.HIGHEST,
                        preferred_element_type=f32)
    r_i = lax.broadcasted_iota(jnp.int32, (64, 512), 0)
    p_i = lax.broadcasted_iota(jnp.int32, (64, 512), 1) // 8
    p2t = (r_i == p_i).astype(f32)
    z1 = lax.dot_general(p2t, y, (((1,), (0,)), ((), ())),
                         precision=lax.Precision.HIGHEST,
                         preferred_element_type=f32)
    q_i = lax.broadcasted_iota(jnp.int32, (GS, 64), 0)
    s_i = lax.broadcasted_iota(jnp.int32, (GS, 64), 1) // 8
    p3t = (q_i == s_i).astype(f32)
    z2 = lax.dot_general(p3t, z1, (((1,), (0,)), ((), ())),
                         precision=lax.Precision.HIGHEST,
                         preferred_element_type=f32)
    out_ref[0] = z2 * (1.0 / (CELL * CELL))


def _topk_kernel(g_ref, out_ref):
    g = g_ref[...]
    b = g.shape[0]
    f32 = jnp.float32
    w = jnp.zeros((b, WS, WS), f32)
    for di in range(WGS):
        for dj in range(WGS):
            w = w + g[:, di:di + WS, dj:dj + WS]
    idx = (WS * lax.broadcasted_iota(jnp.int32, (b, WS, WS), 1)
           + lax.broadcasted_iota(jnp.int32, (b, WS, WS), 2))
    lane = lax.broadcasted_iota(jnp.int32, (b, 2 * TOP_K), 1)
    out = jnp.zeros((b, 2 * TOP_K), jnp.int32)
    neg = jnp.float32(-jnp.inf)
    big = jnp.int32(WS * WS)
    for k in range(TOP_K):
        m = jnp.max(w, axis=(1, 2), keepdims=True)
        cand = jnp.where(w == m, idx, big)
        amin = jnp.min(cand, axis=(1, 2), keepdims=True)
        w = jnp.where(idx == amin, neg, w)
        flat = amin[:, 0, 0]
        row = (flat // WS)[:, None]
        col = (flat % WS)[:, None]
        out = jnp.where(lane == 2 * k, row, out)
        out = jnp.where(lane == 2 * k + 1, col, out)
    out_ref[...] = out


def kernel(sampling_map):
    b, c, h, w = sampling_map.shape
    x = sampling_map.reshape(b, h, w)
    grids = pl.pallas_call(
        _pool_kernel,
        grid=(b,),
        in_specs=[pl.BlockSpec((1, h, w), lambda i: (i, 0, 0))],
        out_specs=pl.BlockSpec((1, GS, GS), lambda i: (i, 0, 0)),
        out_shape=jax.ShapeDtypeStruct((b, GS, GS), jnp.float32),
    )(x)
    coords = pl.pallas_call(
        _topk_kernel,
        out_shape=jax.ShapeDtypeStruct((b, 2 * TOP_K), jnp.int32),
    )(grids)
    return coords.reshape(b, TOP_K, 2)


# speedup vs baseline: 5.0566x; 3.7873x over previous
import functools

import jax
import jax.numpy as jnp
from jax import lax
from jax.experimental import pallas as pl

GS = 8
CELL = 64
WGS = 3
WS = GS - WGS + 1
TOP_K = 4


BB = 8


def _pool_kernel(x_ref, out_ref):
    f32 = jnp.float32
    rows = BB * GS
    t = x_ref[...].reshape(rows, CELL, 512)
    y = jnp.sum(t, axis=1)
    c_i = lax.broadcasted_iota(jnp.int32, (512, 64), 0) // 8
    m_i = lax.broadcasted_iota(jnp.int32, (512, 64), 1)
    pa = (c_i == m_i).astype(f32)
    z1 = lax.dot_general(y, pa, (((1,), (0,)), ((), ())),
                         precision=lax.Precision.HIGHEST,
                         preferred_element_type=f32)
    d_i = lax.broadcasted_iota(jnp.int32, (64, GS), 0) // 8
    j_i = lax.broadcasted_iota(jnp.int32, (64, GS), 1)
    pb = (d_i == j_i).astype(f32)
    z2 = lax.dot_general(z1, pb, (((1,), (0,)), ((), ())),
                         precision=lax.Precision.HIGHEST,
                         preferred_element_type=f32)
    out_ref[...] = z2 * (1.0 / (CELL * CELL))


def _topk_kernel(g_ref, out_ref):
    g = g_ref[...]
    b = g.shape[0]
    f32 = jnp.float32
    w = jnp.zeros((b, WS, WS), f32)
    for di in range(WGS):
        for dj in range(WGS):
            w = w + g[:, di:di + WS, dj:dj + WS]
    idx = (WS * lax.broadcasted_iota(jnp.int32, (b, WS, WS), 1)
           + lax.broadcasted_iota(jnp.int32, (b, WS, WS), 2))
    lane = lax.broadcasted_iota(jnp.int32, (b, 2 * TOP_K), 1)
    out = jnp.zeros((b, 2 * TOP_K), jnp.int32)
    neg = jnp.float32(-jnp.inf)
    big = jnp.int32(WS * WS)
    for k in range(TOP_K):
        m = jnp.max(w, axis=(1, 2), keepdims=True)
        cand = jnp.where(w == m, idx, big)
        amin = jnp.min(cand, axis=(1, 2), keepdims=True)
        w = jnp.where(idx == amin, neg, w)
        flat = amin[:, 0, 0]
        row = (flat // WS)[:, None]
        col = (flat % WS)[:, None]
        out = jnp.where(lane == 2 * k, row, out)
        out = jnp.where(lane == 2 * k + 1, col, out)
    out_ref[...] = out


def kernel(sampling_map):
    b, c, h, w = sampling_map.shape
    x = sampling_map.reshape(b * h, w)
    nsteps = b // BB
    grids = pl.pallas_call(
        _pool_kernel,
        grid=(nsteps,),
        in_specs=[pl.BlockSpec((BB * h, w), lambda i: (i, 0))],
        out_specs=pl.BlockSpec((BB * GS, GS), lambda i: (i, 0)),
        out_shape=jax.ShapeDtypeStruct((b * GS, GS), jnp.float32),
    )(x)
    grids = grids.reshape(b, GS, GS)
    coords = pl.pallas_call(
        _topk_kernel,
        out_shape=jax.ShapeDtypeStruct((b, 2 * TOP_K), jnp.int32),
    )(grids)
    return coords.reshape(b, TOP_K, 2)


# speedup vs baseline: 5.0688x; 1.0024x over previous
import functools

import jax
import jax.numpy as jnp
from jax import lax
from jax.experimental import pallas as pl

GS = 8
CELL = 64
WGS = 3
WS = GS - WGS + 1
TOP_K = 4


BB = 8


def _pool_kernel(x_ref, out_ref):
    f32 = jnp.float32
    rows = BB * GS
    t = x_ref[...].reshape(rows, CELL, 512)
    y = jnp.sum(t, axis=1)
    c_i = lax.broadcasted_iota(jnp.int32, (512, 64), 0) // 8
    m_i = lax.broadcasted_iota(jnp.int32, (512, 64), 1)
    pa = (c_i == m_i).astype(f32)
    z1 = lax.dot_general(y, pa, (((1,), (0,)), ((), ())),
                         precision=lax.Precision.HIGHEST,
                         preferred_element_type=f32)
    d_i = lax.broadcasted_iota(jnp.int32, (64, GS), 0) // 8
    j_i = lax.broadcasted_iota(jnp.int32, (64, GS), 1)
    pb = (d_i == j_i).astype(f32)
    z2 = lax.dot_general(z1, pb, (((1,), (0,)), ((), ())),
                         precision=lax.Precision.HIGHEST,
                         preferred_element_type=f32)
    out_ref[...] = z2 * (1.0 / (CELL * CELL))


def _topk_kernel(g_ref, out_ref):
    g = g_ref[...]
    b = g.shape[0]
    nl = GS * WS
    g = jnp.concatenate([g, jnp.zeros((b, 2 * GS), jnp.float32)], axis=1)
    w = jnp.zeros((b, nl), jnp.float32)
    for di in range(WGS):
        for dj in range(WGS):
            o = GS * di + dj
            w = w + g[:, o:o + nl]
    lane = lax.broadcasted_iota(jnp.int32, (b, nl), 1)
    wi = lane // GS
    wj = lane % GS
    idx = WS * wi + wj
    neg = jnp.float32(-jnp.inf)
    big = jnp.int32(WS * WS)
    w = jnp.where(wj < WS, w, neg)
    lane8 = lax.broadcasted_iota(jnp.int32, (b, 2 * TOP_K), 1)
    out = jnp.zeros((b, 2 * TOP_K), jnp.int32)
    for k in range(TOP_K):
        m = jnp.max(w, axis=1, keepdims=True)
        cand = jnp.where(w == m, idx, big)
        amin = jnp.min(cand, axis=1, keepdims=True)
        w = jnp.where(idx == amin, neg, w)
        row = amin // WS
        col = amin % WS
        out = jnp.where(lane8 == 2 * k, row, out)
        out = jnp.where(lane8 == 2 * k + 1, col, out)
    out_ref[...] = out


def kernel(sampling_map):
    b, c, h, w = sampling_map.shape
    x = sampling_map.reshape(b * h, w)
    nsteps = b // BB
    grids = pl.pallas_call(
        _pool_kernel,
        grid=(nsteps,),
        in_specs=[pl.BlockSpec((BB * h, w), lambda i: (i, 0))],
        out_specs=pl.BlockSpec((BB * GS, GS), lambda i: (i, 0)),
        out_shape=jax.ShapeDtypeStruct((b * GS, GS), jnp.float32),
    )(x)
    grids = grids.reshape(b, GS * GS)
    coords = pl.pallas_call(
        _topk_kernel,
        out_shape=jax.ShapeDtypeStruct((b, 2 * TOP_K), jnp.int32),
    )(grids)
    return coords.reshape(b, TOP_K, 2)
